# baseline (speedup 1.0000x reference)
import functools

import jax
import jax.numpy as jnp
from jax.experimental import pallas as pl
from jax.experimental.pallas import tpu as pltpu
from jax.experimental.pallas import tpu_sc as plsc

_LANES = 16
_W = 32


def kernel(input_ids, wte, wpe):
    b, s = input_ids.shape
    _, e = wte.shape
    n = b * s
    ids_flat = input_ids.reshape(n).astype(jnp.int32)

    mesh = plsc.VectorSubcoreMesh(core_axis_name="c", subcore_axis_name="s")

    @functools.partial(
        pl.kernel,
        out_type=jax.ShapeDtypeStruct((n, e), jnp.float32),
        mesh=mesh,
    )
    def run(wte_hbm, ids_hbm, wpe_hbm, out_hbm):
        def body(ids_v, wpe_v, out_v):
            pltpu.sync_copy(wte_hbm.at[ids_v], out_v)

            @pl.loop(0, _W)
            def _(r):
                @pl.loop(0, e, step=_LANES)
                def _(c):
                    slc = (pl.ds(r, 1), pl.ds(c, _LANES))
                    out_v.at[slc][...] = out_v.at[slc][...] + wpe_v.at[slc][...]

        pltpu.emit_pipeline(
            body,
            grid=(n // _W,),
            in_specs=[
                pl.BlockSpec((_W,), lambda i: (i,)),
                pl.BlockSpec((_W, e), lambda i: (jax.lax.rem(i, s // _W), 0)),
            ],
            out_specs=[pl.BlockSpec((_W, e), lambda i: (i, 0))],
            core_axis_name=("c", "s"),
            dimension_semantics=(pltpu.PARALLEL,),
        )(ids_hbm, wpe_hbm, out_hbm)

    out = run(wte, ids_flat, wpe)
    return out.reshape(b, s, e)


# speedup vs baseline: 1.0658x; 1.0658x over previous
import functools

import jax
import jax.numpy as jnp
from jax.experimental import pallas as pl
from jax.experimental.pallas import tpu as pltpu
from jax.experimental.pallas import tpu_sc as plsc

_LANES = 16
_W = 32


def kernel(input_ids, wte, wpe):
    b, s = input_ids.shape
    _, e = wte.shape
    n = b * s
    ids_flat = input_ids.reshape(n).astype(jnp.int32)

    mesh = plsc.VectorSubcoreMesh(core_axis_name="c", subcore_axis_name="s")

    @functools.partial(
        pl.kernel,
        out_type=jax.ShapeDtypeStruct((n, e), jnp.float32),
        mesh=mesh,
    )
    def run(wte_hbm, ids_hbm, wpe_hbm, out_hbm):
        def body(ids_v, wpe_v, out_v):
            pltpu.sync_copy(wte_hbm.at[ids_v], out_v)

            @pl.loop(0, _W)
            def _(r):
                for c in range(0, e, _LANES):
                    slc = (pl.ds(r, 1), pl.ds(c, _LANES))
                    out_v.at[slc][...] = out_v.at[slc][...] + wpe_v.at[slc][...]

        pltpu.emit_pipeline(
            body,
            grid=(n // _W,),
            in_specs=[
                pl.BlockSpec((_W,), lambda i: (i,)),
                pl.BlockSpec((_W, e), lambda i: (jax.lax.rem(i, s // _W), 0)),
            ],
            out_specs=[pl.BlockSpec((_W, e), lambda i: (i, 0))],
            core_axis_name=("c", "s"),
            dimension_semantics=(pltpu.PARALLEL,),
        )(ids_hbm, wpe_hbm, out_hbm)

    out = run(wte, ids_flat, wpe)
    return out.reshape(b, s, e)


# speedup vs baseline: 1.8446x; 1.7307x over previous
import functools

import jax
import jax.numpy as jnp
from jax import lax
from jax.experimental import pallas as pl
from jax.experimental.pallas import tpu as pltpu
from jax.experimental.pallas import tpu_sc as plsc

_LANES = 16
_NC = 2
_NS = 16
_CH = 16
_RING = 5


def kernel(input_ids, wte, wpe):
    b, s = input_ids.shape
    _, e = wte.shape
    n = b * s
    ids_flat = input_ids.reshape(n).astype(jnp.int32)

    nunits = _NC * _NS
    ppu = s // nunits
    nchunks = ppu // _CH
    nitems = b * nchunks

    mesh = plsc.VectorSubcoreMesh(core_axis_name="c", subcore_axis_name="s")

    scratch = (
        [pltpu.VMEM((b * ppu,), jnp.int32)]
        + [pltpu.VMEM((ppu, e), jnp.float32)]
        + [pltpu.VMEM((_CH, e), jnp.float32) for _ in range(_RING)]
        + [pltpu.SemaphoreType.DMA for _ in range(2 + 2 * _RING)]
    )

    @functools.partial(
        pl.kernel,
        out_type=jax.ShapeDtypeStruct((n, e), jnp.float32),
        mesh=mesh,
        scratch_types=scratch,
    )
    def run(wte_hbm, ids_hbm, wpe_hbm, out_hbm, ids_v, wpe_v, *rest):
        rows = rest[:_RING]
        sem_ids, sem_wpe = rest[_RING], rest[_RING + 1]
        sem_g = rest[_RING + 2:_RING + 2 + _RING]
        sem_o = rest[_RING + 2 + _RING:]

        wid = lax.axis_index("s") * _NC + lax.axis_index("c")
        pos0 = wid * ppu

        h_wpe = pltpu.async_copy(wpe_hbm.at[pl.ds(pos0, ppu)], wpe_v, sem_wpe)

        h_ids = [
            pltpu.async_copy(
                ids_hbm.at[pl.ds(bb * s + pos0, ppu)],
                ids_v.at[pl.ds(bb * ppu, ppu)],
                sem_ids,
            )
            for bb in range(b)
        ]
        for h in h_ids:
            h.wait()

        def fire_gather(j):
            return pltpu.async_copy(
                wte_hbm.at[ids_v.at[pl.ds(j * _CH, _CH)]],
                rows[j % _RING],
                sem_g[j % _RING],
            )

        hg = {}
        ho = {}
        for j in range(_RING - 1):
            hg[j] = fire_gather(j)
        h_wpe.wait()

        for j in range(nitems):
            bb, c = divmod(j, nchunks)
            buf = rows[j % _RING]
            hg[j].wait()

            @pl.loop(0, _CH)
            def _(r):
                for cc in range(0, e, _LANES):
                    slc = (pl.ds(r, 1), pl.ds(cc, _LANES))
                    wslc = (pl.ds(c * _CH + r, 1), pl.ds(cc, _LANES))
                    buf.at[slc][...] = buf.at[slc][...] + wpe_v.at[wslc][...]

            ho[j] = pltpu.async_copy(
                buf,
                out_hbm.at[pl.ds(bb * s + pos0 + c * _CH, _CH)],
                sem_o[j % _RING],
            )
            nxt = j + _RING - 1
            if nxt < nitems:
                if j >= 1:
                    ho[j - 1].wait()
                hg[nxt] = fire_gather(nxt)

        for j in range(max(0, nitems - _RING), nitems):
            if j in ho:
                ho[j].wait()

    out = run(wte, ids_flat, wpe)
    return out.reshape(b, s, e)


# speedup vs baseline: 1.9363x; 1.0497x over previous
import functools

import jax
import jax.numpy as jnp
from jax import lax
from jax.experimental import pallas as pl
from jax.experimental.pallas import tpu as pltpu
from jax.experimental.pallas import tpu_sc as plsc

_LANES = 16
_NC = 2
_NS = 16
_CH = 16
_RING = 6


def kernel(input_ids, wte, wpe):
    b, s = input_ids.shape
    _, e = wte.shape
    n = b * s
    ids_flat = input_ids.reshape(n).astype(jnp.int32)

    nunits = _NC * _NS
    ppu = s // nunits
    nchunks = ppu // _CH
    nitems = b * nchunks

    mesh = plsc.VectorSubcoreMesh(core_axis_name="c", subcore_axis_name="s")

    scratch = (
        [pltpu.VMEM((b * ppu,), jnp.int32)]
        + [pltpu.VMEM((ppu, e), jnp.float32)]
        + [pltpu.VMEM((_CH, e), jnp.float32) for _ in range(_RING)]
        + [pltpu.SemaphoreType.DMA for _ in range(2 + 2 * _RING)]
    )

    @functools.partial(
        pl.kernel,
        out_type=jax.ShapeDtypeStruct((n, e), jnp.float32),
        mesh=mesh,
        scratch_types=scratch,
    )
    def run(wte_hbm, ids_hbm, wpe_hbm, out_hbm, ids_v, wpe_v, *rest):
        rows = rest[:_RING]
        sem_ids, sem_wpe = rest[_RING], rest[_RING + 1]
        sem_g = rest[_RING + 2:_RING + 2 + _RING]
        sem_o = rest[_RING + 2 + _RING:]

        wid = lax.axis_index("s") * _NC + lax.axis_index("c")
        pos0 = wid * ppu

        h_wpe = pltpu.async_copy(wpe_hbm.at[pl.ds(pos0, ppu)], wpe_v, sem_wpe)

        h_ids = [
            pltpu.async_copy(
                ids_hbm.at[pl.ds(bb * s + pos0, ppu)],
                ids_v.at[pl.ds(bb * ppu, ppu)],
                sem_ids,
            )
            for bb in range(b)
        ]
        for h in h_ids:
            h.wait()

        def fire_gather(j):
            return pltpu.async_copy(
                wte_hbm.at[ids_v.at[pl.ds(j * _CH, _CH)]],
                rows[j % _RING],
                sem_g[j % _RING],
            )

        hg = {}
        ho = {}
        for j in range(_RING - 1):
            hg[j] = fire_gather(j)
        h_wpe.wait()

        for j in range(nitems):
            bb, c = divmod(j, nchunks)
            buf = rows[j % _RING]
            hg[j].wait()

            @pl.loop(0, _CH)
            def _(r):
                for cc in range(0, e, _LANES):
                    slc = (pl.ds(r, 1), pl.ds(cc, _LANES))
                    wslc = (pl.ds(c * _CH + r, 1), pl.ds(cc, _LANES))
                    plsc.addupdate(buf.at[slc], wpe_v.at[wslc][...])

            ho[j] = pltpu.async_copy(
                buf,
                out_hbm.at[pl.ds(bb * s + pos0 + c * _CH, _CH)],
                sem_o[j % _RING],
            )
            nxt = j + _RING - 1
            if nxt < nitems:
                if j >= 1:
                    ho[j - 1].wait()
                hg[nxt] = fire_gather(nxt)

        for j in range(max(0, nitems - _RING), nitems):
            if j in ho:
                ho[j].wait()

    out = run(wte, ids_flat, wpe)
    return out.reshape(b, s, e)


# speedup vs baseline: 2.0090x; 1.0375x over previous
import functools

import jax
import jax.numpy as jnp
from jax import lax
from jax.experimental import pallas as pl
from jax.experimental.pallas import tpu as pltpu
from jax.experimental.pallas import tpu_sc as plsc

_LANES = 16
_NC = 2
_NS = 16
_CH = 32
_RING = 3


def kernel(input_ids, wte, wpe):
    b, s = input_ids.shape
    _, e = wte.shape
    n = b * s
    ids_flat = input_ids.reshape(n).astype(jnp.int32)

    nunits = _NC * _NS
    ppu = s // nunits
    nchunks = ppu // _CH
    nitems = b * nchunks

    mesh = plsc.VectorSubcoreMesh(core_axis_name="c", subcore_axis_name="s")

    scratch = (
        [pltpu.VMEM((b * ppu,), jnp.int32)]
        + [pltpu.VMEM((ppu, e), jnp.float32)]
        + [pltpu.VMEM((_CH, e), jnp.float32) for _ in range(_RING)]
        + [pltpu.SemaphoreType.DMA for _ in range(2 + 2 * _RING)]
    )

    @functools.partial(
        pl.kernel,
        out_type=jax.ShapeDtypeStruct((n, e), jnp.float32),
        mesh=mesh,
        scratch_types=scratch,
    )
    def run(wte_hbm, ids_hbm, wpe_hbm, out_hbm, ids_v, wpe_v, *rest):
        rows = rest[:_RING]
        sem_ids, sem_wpe = rest[_RING], rest[_RING + 1]
        sem_g = rest[_RING + 2:_RING + 2 + _RING]
        sem_o = rest[_RING + 2 + _RING:]

        wid = lax.axis_index("s") * _NC + lax.axis_index("c")
        pos0 = wid * ppu

        h_wpe = pltpu.async_copy(wpe_hbm.at[pl.ds(pos0, ppu)], wpe_v, sem_wpe)

        h_ids = [
            pltpu.async_copy(
                ids_hbm.at[pl.ds(bb * s + pos0, ppu)],
                ids_v.at[pl.ds(bb * ppu, ppu)],
                sem_ids,
            )
            for bb in range(b)
        ]
        for h in h_ids:
            h.wait()

        def fire_gather(j):
            return pltpu.async_copy(
                wte_hbm.at[ids_v.at[pl.ds(j * _CH, _CH)]],
                rows[j % _RING],
                sem_g[j % _RING],
            )

        hg = {}
        ho = {}
        for j in range(_RING - 1):
            hg[j] = fire_gather(j)
        h_wpe.wait()

        for j in range(nitems):
            bb, c = divmod(j, nchunks)
            buf = rows[j % _RING]
            hg[j].wait()

            @pl.loop(0, _CH)
            def _(r):
                for cc in range(0, e, _LANES):
                    slc = (pl.ds(r, 1), pl.ds(cc, _LANES))
                    wslc = (pl.ds(c * _CH + r, 1), pl.ds(cc, _LANES))
                    plsc.addupdate(buf.at[slc], wpe_v.at[wslc][...])

            ho[j] = pltpu.async_copy(
                buf,
                out_hbm.at[pl.ds(bb * s + pos0 + c * _CH, _CH)],
                sem_o[j % _RING],
            )
            nxt = j + _RING - 1
            if nxt < nitems:
                if j >= 1:
                    ho[j - 1].wait()
                hg[nxt] = fire_gather(nxt)

        for j in range(max(0, nitems - _RING), nitems):
            if j in ho:
                ho[j].wait()

    out = run(wte, ids_flat, wpe)
    return out.reshape(b, s, e)


# speedup vs baseline: 2.3399x; 1.1647x over previous
import functools

import jax
import jax.numpy as jnp
from jax import lax
from jax.experimental import pallas as pl
from jax.experimental.pallas import tpu as pltpu
from jax.experimental.pallas import tpu_sc as plsc

_LANES = 16
_NC = 2
_NS = 16
_CH = 32
_RING = 3


def kernel(input_ids, wte, wpe):
    b, s = input_ids.shape
    _, e = wte.shape
    n = b * s
    ids_flat = input_ids.reshape(n).astype(jnp.int32)

    nunits = _NC * _NS
    ppu = s // nunits
    nchunks = ppu // _CH
    nitems = b * nchunks

    mesh = plsc.VectorSubcoreMesh(core_axis_name="c", subcore_axis_name="s")

    scratch = (
        [pltpu.VMEM((b * ppu,), jnp.int32)]
        + [pltpu.VMEM((ppu, e), jnp.float32)]
        + [pltpu.VMEM((_CH, e), jnp.float32) for _ in range(_RING)]
        + [pltpu.SemaphoreType.DMA for _ in range(2 + 2 * _RING)]
    )

    @functools.partial(
        pl.kernel,
        out_type=jax.ShapeDtypeStruct((n, e), jnp.float32),
        mesh=mesh,
        scratch_types=scratch,
    )
    def run(wte_hbm, ids_hbm, wpe_hbm, out_hbm, ids_v, wpe_v, *rest):
        rows = rest[:_RING]
        sem_ids, sem_wpe = rest[_RING], rest[_RING + 1]
        sem_g = rest[_RING + 2:_RING + 2 + _RING]
        sem_o = rest[_RING + 2 + _RING:]

        wid = lax.axis_index("s") * _NC + lax.axis_index("c")
        pos0 = wid * ppu

        h_wpe = pltpu.async_copy(wpe_hbm.at[pl.ds(pos0, ppu)], wpe_v, sem_wpe)

        h_ids = [
            pltpu.async_copy(
                ids_hbm.at[pl.ds(bb * s + pos0, ppu)],
                ids_v.at[pl.ds(bb * ppu, ppu)],
                sem_ids,
            )
            for bb in range(b)
        ]
        for h in h_ids:
            h.wait()

        def fire_gather(j):
            return pltpu.async_copy(
                wte_hbm.at[ids_v.at[pl.ds(j * _CH, _CH)]],
                rows[j % _RING],
                sem_g[j % _RING],
            )

        hg = {}
        ho = {}
        for j in range(_RING - 1):
            hg[j] = fire_gather(j)
        h_wpe.wait()

        for j in range(nitems):
            bb, c = divmod(j, nchunks)
            buf = rows[j % _RING]
            hg[j].wait()

            @plsc.parallel_loop(0, _CH)
            def _(r):
                for cc in range(0, e, _LANES):
                    slc = (pl.ds(r, 1), pl.ds(cc, _LANES))
                    wslc = (pl.ds(c * _CH + r, 1), pl.ds(cc, _LANES))
                    plsc.addupdate(buf.at[slc], wpe_v.at[wslc][...])

            ho[j] = pltpu.async_copy(
                buf,
                out_hbm.at[pl.ds(bb * s + pos0 + c * _CH, _CH)],
                sem_o[j % _RING],
            )
            nxt = j + _RING - 1
            if nxt < nitems:
                if j >= 1:
                    ho[j - 1].wait()
                hg[nxt] = fire_gather(nxt)

        for j in range(max(0, nitems - _RING), nitems):
            if j in ho:
                ho[j].wait()

    out = run(wte, ids_flat, wpe)
    return out.reshape(b, s, e)
